# Initial kernel scaffold; baseline (speedup 1.0000x reference)
import jax
import jax.numpy as jnp
from jax.experimental import pallas as pl


def kernel(input_ids, table):
    raise NotImplementedError("write your pallas kernel here")


# speedup vs baseline: 1.5332x; 1.5332x over previous
import functools

import jax
import jax.numpy as jnp
from jax import lax
from jax.experimental import pallas as pl
from jax.experimental.pallas import tpu as pltpu
from jax.experimental.pallas import tpu_sc as plsc

_NUM_CORES = 2
_NUM_SUBCORES = 16
_NUM_WORKERS = _NUM_CORES * _NUM_SUBCORES

_CHUNK = 32


@functools.partial(jax.jit, static_argnames=())
def _gather_rows(ids, table):
    B = ids.shape[0]
    V, D = table.shape
    b_per_w = B // _NUM_WORKERS
    n_chunks = b_per_w // _CHUNK

    mesh = plsc.VectorSubcoreMesh(core_axis_name="c", subcore_axis_name="s")

    @functools.partial(
        pl.kernel,
        mesh=mesh,
        out_type=jax.ShapeDtypeStruct((B, D), jnp.float32),
        scratch_types=[
            pltpu.VMEM((_CHUNK,), jnp.int32),
            pltpu.VMEM((_CHUNK, D), jnp.float32),
            pltpu.SemaphoreType.DMA,
        ],
    )
    def body(ids_hbm, table_hbm, out_hbm, idx_v, rows_v, sem):
        wid = lax.axis_index("s") * _NUM_CORES + lax.axis_index("c")
        base = wid * b_per_w

        def step(c, carry):
            off = base + c * _CHUNK
            pltpu.sync_copy(ids_hbm.at[pl.ds(off, _CHUNK)], idx_v)
            pltpu.async_copy(table_hbm.at[idx_v], rows_v, sem).wait()
            pltpu.sync_copy(rows_v, out_hbm.at[pl.ds(off, _CHUNK)])
            return carry

        lax.fori_loop(0, n_chunks, step, 0, unroll=False)

    return body(ids, table)


def kernel(input_ids, table):
    ids = input_ids.reshape(-1).astype(jnp.int32)
    out = _gather_rows(ids, table)
    return out.reshape(input_ids.shape + (table.shape[1],))


# speedup vs baseline: 1.6210x; 1.0573x over previous
import functools

import jax
import jax.numpy as jnp
from jax import lax
from jax.experimental import pallas as pl
from jax.experimental.pallas import tpu as pltpu
from jax.experimental.pallas import tpu_sc as plsc

_NUM_CORES = 2
_NUM_SUBCORES = 16
_NUM_WORKERS = _NUM_CORES * _NUM_SUBCORES

_CHUNK = 16


@jax.jit
def _gather_rows(ids, table):
    B = ids.shape[0]
    V, D = table.shape
    b_per_w = B // _NUM_WORKERS
    n_chunks = b_per_w // _CHUNK

    mesh = plsc.VectorSubcoreMesh(core_axis_name="c", subcore_axis_name="s")

    @functools.partial(
        pl.kernel,
        mesh=mesh,
        out_type=jax.ShapeDtypeStruct((B, D), jnp.float32),
        scratch_types=[
            pltpu.VMEM((b_per_w,), jnp.int32),
            pltpu.VMEM((_CHUNK, D), jnp.float32),
            pltpu.VMEM((_CHUNK, D), jnp.float32),
            pltpu.SemaphoreType.DMA,
            pltpu.SemaphoreType.DMA,
        ],
    )
    def body(ids_hbm, table_hbm, out_hbm, idx_v, buf_a, buf_b, gsem, wsem):
        wid = lax.axis_index("s") * _NUM_CORES + lax.axis_index("c")
        base = wid * b_per_w
        pltpu.sync_copy(ids_hbm.at[pl.ds(base, b_per_w)], idx_v)

        def idx_at(c):
            return idx_v.at[pl.ds(c * _CHUNK, _CHUNK)]

        def out_at(c):
            return out_hbm.at[pl.ds(base + c * _CHUNK, _CHUNK)]

        def gather_start(c, buf):
            pltpu.async_copy(table_hbm.at[idx_at(c)], buf, gsem)

        def gather_wait(c, buf):
            pltpu.make_async_copy(table_hbm.at[idx_at(c)], buf, gsem).wait()

        def write_start(c, buf):
            pltpu.async_copy(buf, out_at(c), wsem)

        def write_wait(c, buf):
            pltpu.make_async_copy(buf, out_at(c), wsem).wait()

        def step(c2, carry):
            c0 = 2 * c2
            c1 = c0 + 1
            gather_start(c0, buf_a)

            @pl.when(c2 > 0)
            def _():
                write_wait(c1 - 2, buf_b)

            gather_wait(c0, buf_a)
            write_start(c0, buf_a)
            gather_start(c1, buf_b)
            write_wait(c0, buf_a)
            gather_wait(c1, buf_b)
            write_start(c1, buf_b)
            return carry

        lax.fori_loop(0, n_chunks // 2, step, 0, unroll=False)
        write_wait(n_chunks - 1, buf_b)

    return body(ids, table)


def kernel(input_ids, table):
    ids = input_ids.reshape(-1).astype(jnp.int32)
    out = _gather_rows(ids, table)
    return out.reshape(input_ids.shape + (table.shape[1],))


# speedup vs baseline: 1.6770x; 1.0346x over previous
import functools

import jax
import jax.numpy as jnp
from jax import lax
from jax.experimental import pallas as pl
from jax.experimental.pallas import tpu as pltpu
from jax.experimental.pallas import tpu_sc as plsc

_NUM_CORES = 2
_NUM_SUBCORES = 16
_NUM_WORKERS = _NUM_CORES * _NUM_SUBCORES

_CHUNK = 16


@jax.jit
def _gather_rows(ids, table):
    B = ids.shape[0]
    V, D = table.shape
    b_per_w = B // _NUM_WORKERS
    n_chunks = b_per_w // _CHUNK

    mesh = plsc.VectorSubcoreMesh(core_axis_name="c", subcore_axis_name="s")

    @functools.partial(
        pl.kernel,
        mesh=mesh,
        out_type=jax.ShapeDtypeStruct((B, D), jnp.float32),
        scratch_types=[
            pltpu.VMEM((b_per_w,), jnp.int32),
            pltpu.VMEM((3 * _CHUNK, D), jnp.float32),
            pltpu.SemaphoreType.DMA,
            pltpu.SemaphoreType.DMA,
        ],
    )
    def body(ids_hbm, table_hbm, out_hbm, idx_v, bufs, gsem, wsem):
        wid = lax.axis_index("s") * _NUM_CORES + lax.axis_index("c")
        base = wid * b_per_w
        pltpu.sync_copy(ids_hbm.at[pl.ds(base, b_per_w)], idx_v)

        def idx_at(c):
            return idx_v.at[pl.ds(c * _CHUNK, _CHUNK)]

        def out_at(c):
            return out_hbm.at[pl.ds(base + c * _CHUNK, _CHUNK)]

        def buf_at(c):
            return bufs.at[pl.ds(lax.rem(c, 3) * _CHUNK, _CHUNK)]

        def gather_start(c):
            pltpu.async_copy(table_hbm.at[idx_at(c)], buf_at(c), gsem)

        def gather_wait(c):
            pltpu.make_async_copy(table_hbm.at[idx_at(c)], buf_at(c), gsem).wait()

        def write_start(c):
            pltpu.async_copy(buf_at(c), out_at(c), wsem)

        def write_wait(c):
            pltpu.make_async_copy(buf_at(c), out_at(c), wsem).wait()

        gather_start(0)
        gather_start(1)

        def step(c, carry):
            @pl.when(c >= 1)
            def _():
                write_wait(c - 1)

            @pl.when(c + 2 < n_chunks)
            def _():
                gather_start(c + 2)

            gather_wait(c)
            write_start(c)
            return carry

        lax.fori_loop(0, n_chunks, step, 0, unroll=False)
        write_wait(n_chunks - 1)

    return body(ids, table)


def kernel(input_ids, table):
    ids = input_ids.reshape(-1).astype(jnp.int32)
    out = _gather_rows(ids, table)
    return out.reshape(input_ids.shape + (table.shape[1],))


# speedup vs baseline: 1.6857x; 1.0052x over previous
import functools

import jax
import jax.numpy as jnp
from jax import lax
from jax.experimental import pallas as pl
from jax.experimental.pallas import tpu as pltpu
from jax.experimental.pallas import tpu_sc as plsc

_NUM_CORES = 2
_NUM_SUBCORES = 16
_NUM_WORKERS = _NUM_CORES * _NUM_SUBCORES

_CHUNK = 8
_SLOTS = 6


@jax.jit
def _gather_rows(ids, table):
    B = ids.shape[0]
    V, D = table.shape
    b_per_w = B // _NUM_WORKERS
    n_chunks = b_per_w // _CHUNK

    mesh = plsc.VectorSubcoreMesh(core_axis_name="c", subcore_axis_name="s")

    @functools.partial(
        pl.kernel,
        mesh=mesh,
        out_type=jax.ShapeDtypeStruct((B, D), jnp.float32),
        scratch_types=[
            pltpu.VMEM((b_per_w,), jnp.int32),
            pltpu.VMEM((_SLOTS * _CHUNK, D), jnp.float32),
            pltpu.SemaphoreType.DMA,
            pltpu.SemaphoreType.DMA,
        ],
    )
    def body(ids_hbm, table_hbm, out_hbm, idx_v, bufs, gsem, wsem):
        wid = lax.axis_index("s") * _NUM_CORES + lax.axis_index("c")
        base = wid * b_per_w
        pltpu.sync_copy(ids_hbm.at[pl.ds(base, b_per_w)], idx_v)

        def idx_at(c):
            return idx_v.at[pl.ds(c * _CHUNK, _CHUNK)]

        def out_at(c):
            return out_hbm.at[pl.ds(base + c * _CHUNK, _CHUNK)]

        def buf_at(c):
            return bufs.at[pl.ds(lax.rem(c, _SLOTS) * _CHUNK, _CHUNK)]

        def gather_start(c):
            pltpu.async_copy(table_hbm.at[idx_at(c)], buf_at(c), gsem)

        def gather_wait(c):
            pltpu.make_async_copy(table_hbm.at[idx_at(c)], buf_at(c), gsem).wait()

        def write_start(c):
            pltpu.async_copy(buf_at(c), out_at(c), wsem)

        def write_wait(c):
            pltpu.make_async_copy(buf_at(c), out_at(c), wsem).wait()

        gather_start(0)
        gather_start(1)

        def step(c, carry):
            @pl.when(c >= 3)
            def _():
                write_wait(c - 3)

            @pl.when(c + 2 < n_chunks)
            def _():
                gather_start(c + 2)

            gather_wait(c)
            write_start(c)
            return carry

        lax.fori_loop(0, n_chunks, step, 0, unroll=False)
        write_wait(n_chunks - 3)
        write_wait(n_chunks - 2)
        write_wait(n_chunks - 1)

    return body(ids, table)


def kernel(input_ids, table):
    ids = input_ids.reshape(-1).astype(jnp.int32)
    out = _gather_rows(ids, table)
    return out.reshape(input_ids.shape + (table.shape[1],))


# speedup vs baseline: 1.6876x; 1.0011x over previous
import functools

import jax
import jax.numpy as jnp
from jax import lax
from jax.experimental import pallas as pl
from jax.experimental.pallas import tpu as pltpu
from jax.experimental.pallas import tpu_sc as plsc

_NUM_CORES = 2
_NUM_SUBCORES = 16
_NUM_WORKERS = _NUM_CORES * _NUM_SUBCORES

_CHUNK = 8
_SLOTS = 7


@jax.jit
def _gather_rows(ids, table):
    B = ids.shape[0]
    V, D = table.shape
    b_per_w = B // _NUM_WORKERS
    n_chunks = b_per_w // _CHUNK

    mesh = plsc.VectorSubcoreMesh(core_axis_name="c", subcore_axis_name="s")

    @functools.partial(
        pl.kernel,
        mesh=mesh,
        out_type=jax.ShapeDtypeStruct((B, D), jnp.float32),
        scratch_types=[
            pltpu.VMEM((b_per_w,), jnp.int32),
            pltpu.VMEM((_SLOTS * _CHUNK, D), jnp.float32),
            pltpu.SemaphoreType.DMA,
            pltpu.SemaphoreType.DMA,
        ],
    )
    def body(ids_hbm, table_hbm, out_hbm, idx_v, bufs, gsem, wsem):
        wid = lax.axis_index("s") * _NUM_CORES + lax.axis_index("c")
        base = wid * b_per_w
        pltpu.sync_copy(ids_hbm.at[pl.ds(base, b_per_w)], idx_v)

        def idx_at(c):
            return idx_v.at[pl.ds(c * _CHUNK, _CHUNK)]

        def out_at(c):
            return out_hbm.at[pl.ds(base + c * _CHUNK, _CHUNK)]

        def buf_at(c):
            return bufs.at[pl.ds(lax.rem(c, _SLOTS) * _CHUNK, _CHUNK)]

        def gather_start(c):
            pltpu.async_copy(table_hbm.at[idx_at(c)], buf_at(c), gsem)

        def gather_wait(c):
            pltpu.make_async_copy(table_hbm.at[idx_at(c)], buf_at(c), gsem).wait()

        def write_start(c):
            pltpu.async_copy(buf_at(c), out_at(c), wsem)

        def write_wait(c):
            pltpu.make_async_copy(buf_at(c), out_at(c), wsem).wait()

        gather_start(0)
        gather_start(1)

        def step(c, carry):
            @pl.when(c >= 4)
            def _():
                write_wait(c - 4)

            @pl.when(c + 2 < n_chunks)
            def _():
                gather_start(c + 2)

            gather_wait(c)
            write_start(c)
            return carry

        lax.fori_loop(0, n_chunks, step, 0, unroll=False)
        write_wait(n_chunks - 4)
        write_wait(n_chunks - 3)
        write_wait(n_chunks - 2)
        write_wait(n_chunks - 1)

    return body(ids, table)


def kernel(input_ids, table):
    ids = input_ids.reshape(-1).astype(jnp.int32)
    out = _gather_rows(ids, table)
    return out.reshape(input_ids.shape + (table.shape[1],))


# speedup vs baseline: 1.6953x; 1.0046x over previous
import functools

import jax
import jax.numpy as jnp
from jax import lax
from jax.experimental import pallas as pl
from jax.experimental.pallas import tpu as pltpu
from jax.experimental.pallas import tpu_sc as plsc

_NUM_CORES = 2
_NUM_SUBCORES = 16
_NUM_WORKERS = _NUM_CORES * _NUM_SUBCORES

_CHUNK = 8
_SLOTS = 7


def _embed(ids, table):
    BATCH, SEQ = ids.shape
    V, D = table.shape
    B = BATCH * SEQ
    b_per_w = B // _NUM_WORKERS
    n_chunks = b_per_w // _CHUNK

    mesh = plsc.VectorSubcoreMesh(core_axis_name="c", subcore_axis_name="s")

    @functools.partial(
        pl.kernel,
        mesh=mesh,
        out_type=jax.ShapeDtypeStruct((BATCH, SEQ, D), jnp.float32),
        scratch_types=[
            pltpu.VMEM((b_per_w,), jnp.int32),
            pltpu.VMEM((_SLOTS * _CHUNK, D), jnp.float32),
            pltpu.SemaphoreType.DMA,
            pltpu.SemaphoreType.DMA,
        ],
    )
    def body(ids_hbm, table_hbm, out_hbm, idx_v, bufs, gsem, wsem):
        wid = lax.axis_index("s") * _NUM_CORES + lax.axis_index("c")
        base = wid * b_per_w
        b = base // SEQ
        col = base - b * SEQ
        pltpu.sync_copy(ids_hbm.at[b, pl.ds(col, b_per_w)], idx_v)

        def idx_at(c):
            return idx_v.at[pl.ds(c * _CHUNK, _CHUNK)]

        def out_at(c):
            return out_hbm.at[b, pl.ds(col + c * _CHUNK, _CHUNK)]

        def buf_at(c):
            return bufs.at[pl.ds(lax.rem(c, _SLOTS) * _CHUNK, _CHUNK)]

        def gather_start(c):
            pltpu.async_copy(table_hbm.at[idx_at(c)], buf_at(c), gsem)

        def gather_wait(c):
            pltpu.make_async_copy(table_hbm.at[idx_at(c)], buf_at(c), gsem).wait()

        def write_start(c):
            pltpu.async_copy(buf_at(c), out_at(c), wsem)

        def write_wait(c):
            pltpu.make_async_copy(buf_at(c), out_at(c), wsem).wait()

        gather_start(0)
        gather_start(1)

        def step(c, carry):
            @pl.when(c >= 4)
            def _():
                write_wait(c - 4)

            @pl.when(c + 2 < n_chunks)
            def _():
                gather_start(c + 2)

            gather_wait(c)
            write_start(c)
            return carry

        lax.fori_loop(0, n_chunks, step, 0, unroll=False)
        write_wait(n_chunks - 4)
        write_wait(n_chunks - 3)
        write_wait(n_chunks - 2)
        write_wait(n_chunks - 1)

    return body(ids, table)


def kernel(input_ids, table):
    return _embed(input_ids.astype(jnp.int32), table)
